# baseline (speedup 1.0000x reference)
import dataclasses
import functools

import jax
import jax.numpy as jnp
from jax import lax
from jax.experimental import pallas as pl
from jax.experimental.pallas import tpu as pltpu
from jax.experimental.pallas import tpu_sc as plsc

D = 200
DP = 256
NC = 2
NS = 16
L = 16
NW = NC * NS
W = 32
G = W // L


def _rsqrt(x):
    i = plsc.bitcast(x, jnp.int32)
    i = jnp.int32(0x5F3759DF) - (i >> 1)
    y = plsc.bitcast(i, jnp.float32)
    for _ in range(3):
        y = y * (jnp.float32(1.5) - jnp.float32(0.5) * x * y * y)
    return y


def _build_dist_kernel(tot):
    rpt = tot // NW
    ch = rpt // W
    mesh = plsc.VectorSubcoreMesh(core_axis_name="c", subcore_axis_name="s")
    cp = pltpu.CompilerParams()
    if "needs_layout_passes" in pltpu.CompilerParams.__dataclass_fields__:
        cp = dataclasses.replace(cp, needs_layout_passes=False)
    if "use_tc_tiling_on_sc" in pltpu.CompilerParams.__dataclass_fields__:
        cp = dataclasses.replace(cp, use_tc_tiling_on_sc=False)

    @functools.partial(
        pl.kernel,
        mesh=mesh,
        compiler_params=cp,
        out_type=jax.ShapeDtypeStruct((tot,), jnp.float32),
        scratch_types=[
            pltpu.VMEM((rpt,), jnp.int32),
            pltpu.VMEM((rpt,), jnp.int32),
            pltpu.VMEM((rpt,), jnp.int32),
            pltpu.VMEM((W, D), jnp.float32),
            pltpu.VMEM((W, D), jnp.float32),
            pltpu.VMEM((W, D), jnp.float32),
            pltpu.VMEM((W, D), jnp.float32),
            pltpu.VMEM((W, D), jnp.float32),
            pltpu.VMEM((W, D), jnp.float32),
            pltpu.VMEM((rpt,), jnp.float32),
            pltpu.SemaphoreType.DMA,
        ],
    )
    def dist_kernel(ent_e_hbm, rel_e_hbm, ent_t_hbm, rel_t_hbm,
                    h_hbm, t_hbm, r_hbm, out_hbm,
                    hi, ti, ri, he, ht, te, tt, re, rt, res, sem):
        wid = lax.axis_index("s") * NC + lax.axis_index("c")
        base = wid * rpt
        pltpu.sync_copy(h_hbm.at[pl.ds(base, rpt)], hi)
        pltpu.sync_copy(t_hbm.at[pl.ds(base, rpt)], ti)
        pltpu.sync_copy(r_hbm.at[pl.ds(base, rpt)], ri)

        @pl.loop(0, ch)
        def _chunk(c):
            off = pl.multiple_of(c * W, W)
            dmas = [
                pltpu.async_copy(ent_e_hbm.at[hi.at[pl.ds(off, W)]], he, sem),
                pltpu.async_copy(ent_t_hbm.at[hi.at[pl.ds(off, W)]], ht, sem),
                pltpu.async_copy(ent_e_hbm.at[ti.at[pl.ds(off, W)]], te, sem),
                pltpu.async_copy(ent_t_hbm.at[ti.at[pl.ds(off, W)]], tt, sem),
                pltpu.async_copy(rel_e_hbm.at[ri.at[pl.ds(off, W)]], re, sem),
                pltpu.async_copy(rel_t_hbm.at[ri.at[pl.ds(off, W)]], rt, sem),
            ]
            for dma in dmas:
                dma.wait()

            for g in range(G):
                rows = lax.iota(jnp.int32, L) + jnp.int32(g * L)
                z = jnp.zeros((L,), jnp.float32)

                def pass_a(d, carry):
                    sh, st, ah, at_, chv, ctv, qv = carry
                    cd = jnp.full((L,), d, jnp.int32)
                    hev = plsc.load_gather(he, [rows, cd])
                    htv = plsc.load_gather(ht, [rows, cd])
                    tev = plsc.load_gather(te, [rows, cd])
                    ttv = plsc.load_gather(tt, [rows, cd])
                    rtv = plsc.load_gather(rt, [rows, cd])
                    return (sh + hev * htv, st + tev * ttv,
                            ah + hev * hev, at_ + tev * tev,
                            chv + hev * rtv, ctv + tev * rtv,
                            qv + rtv * rtv)

                sh, st, ah, at_, chv, ctv, qv = lax.fori_loop(
                    0, D, pass_a, (z, z, z, z, z, z, z))

                two = jnp.float32(2.0)
                nh = ah + two * sh * chv + sh * sh * qv
                nt = at_ + two * st * ctv + st * st * qv
                eps = jnp.float32(1e-12)
                ih = _rsqrt(jnp.maximum(nh, eps))
                it = _rsqrt(jnp.maximum(nt, eps))

                def pass_c(d, acc):
                    cd = jnp.full((L,), d, jnp.int32)
                    hev = plsc.load_gather(he, [rows, cd])
                    tev = plsc.load_gather(te, [rows, cd])
                    rtv = plsc.load_gather(rt, [rows, cd])
                    rev = plsc.load_gather(re, [rows, cd])
                    ph = (hev + sh * rtv) * ih
                    pt = (tev + st * rtv) * it
                    return acc + jnp.abs(ph + rev - pt)

                dv = lax.fori_loop(0, D, pass_c, z)
                res[pl.ds(off + g * L, L)] = dv

        pltpu.sync_copy(res, out_hbm.at[pl.ds(base, rpt)])

    return dist_kernel


def kernel(ent_emb, rel_emb, ent_transfer, rel_transfer,
           pos_h_id, pos_t_id, pos_r_id, neg_h_id, neg_t_id, neg_r_id):
    b = pos_h_id.shape[0]
    h_id = jnp.concatenate([pos_h_id, neg_h_id]).astype(jnp.int32)
    t_id = jnp.concatenate([pos_t_id, neg_t_id]).astype(jnp.int32)
    r_id = jnp.concatenate([pos_r_id, neg_r_id]).astype(jnp.int32)
    def lin(x):
        return lax.optimization_barrier(x.reshape(-1)).reshape(x.shape)
    dist = _build_dist_kernel(2 * b)(
        lin(ent_emb), lin(rel_emb), lin(ent_transfer), lin(rel_transfer),
        h_id, t_id, r_id)
    return dist[:b, None], dist[b:, None]


# speedup vs baseline: 2.3020x; 2.3020x over previous
import dataclasses
import functools

import jax
import jax.numpy as jnp
from jax import lax
from jax.experimental import pallas as pl
from jax.experimental.pallas import tpu as pltpu
from jax.experimental.pallas import tpu_sc as plsc

D = 200
DP = 256
NC = 2
NS = 16
L = 16
NW = NC * NS
W = 32
CHUNK_OFFS = tuple(range(0, D - L + 1, L)) + (D - L,)


def _pad_table(x):
    n = x.shape[0]
    bk = n // 10 if n % 10 == 0 and (n // 10) % 8 == 0 else n

    def body(x_ref, o_ref):
        o_ref[:, :D] = x_ref[...]
        o_ref[:, D:] = jnp.zeros((bk, DP - D), x.dtype)

    return pl.pallas_call(
        body,
        grid=(n // bk,),
        in_specs=[pl.BlockSpec((bk, D), lambda i: (i, 0))],
        out_specs=pl.BlockSpec((bk, DP), lambda i: (i, 0)),
        out_shape=jax.ShapeDtypeStruct((n, DP), x.dtype),
    )(x)


def _rsqrt(x):
    i = lax.bitcast_convert_type(x, jnp.int32)
    i = jnp.int32(0x5F3759DF) - (i >> 1)
    y = lax.bitcast_convert_type(i, jnp.float32)
    for _ in range(3):
        y = y * (jnp.float32(1.5) - jnp.float32(0.5) * x * y * y)
    return y


def _build_dist_kernel(tot):
    rpt = tot // NW
    ch = rpt // W
    mesh = plsc.VectorSubcoreMesh(core_axis_name="c", subcore_axis_name="s")
    cp = pltpu.CompilerParams()
    if "needs_layout_passes" in pltpu.CompilerParams.__dataclass_fields__:
        cp = dataclasses.replace(cp, needs_layout_passes=False)

    @functools.partial(
        pl.kernel,
        mesh=mesh,
        compiler_params=cp,
        out_type=jax.ShapeDtypeStruct((tot,), jnp.float32),
        scratch_types=[
            pltpu.VMEM((rpt,), jnp.int32),
            pltpu.VMEM((rpt,), jnp.int32),
            pltpu.VMEM((rpt,), jnp.int32),
            pltpu.VMEM((W, DP), jnp.float32),
            pltpu.VMEM((W, DP), jnp.float32),
            pltpu.VMEM((W, DP), jnp.float32),
            pltpu.VMEM((W, DP), jnp.float32),
            pltpu.VMEM((W, DP), jnp.float32),
            pltpu.VMEM((W, DP), jnp.float32),
            pltpu.VMEM((rpt,), jnp.float32),
            pltpu.SemaphoreType.DMA,
        ],
    )
    def dist_kernel(ent_e_hbm, rel_e_hbm, ent_t_hbm, rel_t_hbm,
                    h_hbm, t_hbm, r_hbm, out_hbm,
                    hi, ti, ri, he, ht, te, tt, re, rt, res, sem):
        wid = lax.axis_index("s") * NC + lax.axis_index("c")
        base = wid * rpt
        pltpu.sync_copy(h_hbm.at[pl.ds(base, rpt)], hi)
        pltpu.sync_copy(t_hbm.at[pl.ds(base, rpt)], ti)
        pltpu.sync_copy(r_hbm.at[pl.ds(base, rpt)], ri)

        tail_mask = jnp.where(lax.iota(jnp.int32, L) < jnp.int32(L - D % L),
                              jnp.float32(0.0), jnp.float32(1.0))

        @pl.loop(0, ch)
        def _chunk(c):
            off = pl.multiple_of(c * W, W)
            dmas = [
                pltpu.async_copy(ent_e_hbm.at[hi.at[pl.ds(off, W)]], he, sem),
                pltpu.async_copy(ent_t_hbm.at[hi.at[pl.ds(off, W)]], ht, sem),
                pltpu.async_copy(ent_e_hbm.at[ti.at[pl.ds(off, W)]], te, sem),
                pltpu.async_copy(ent_t_hbm.at[ti.at[pl.ds(off, W)]], tt, sem),
                pltpu.async_copy(rel_e_hbm.at[ri.at[pl.ds(off, W)]], re, sem),
                pltpu.async_copy(rel_t_hbm.at[ri.at[pl.ds(off, W)]], rt, sem),
            ]
            for dma in dmas:
                dma.wait()

            lane_ids = lax.iota(jnp.int32, L)

            def _row(w, dacc):
                z = jnp.zeros((L,), jnp.float32)
                sh = st = ah = at_ = chv = ctv = qv = z
                for k, o in enumerate(CHUNK_OFFS):
                    sl = (w, pl.ds(o, L))
                    hev, htv = he[sl], ht[sl]
                    tev, ttv = te[sl], tt[sl]
                    rtv = rt[sl]
                    if k == len(CHUNK_OFFS) - 1:
                        hev = hev * tail_mask
                        tev = tev * tail_mask
                        rtv = rtv * tail_mask
                    sh = sh + hev * htv
                    st = st + tev * ttv
                    ah = ah + hev * hev
                    at_ = at_ + tev * tev
                    chv = chv + hev * rtv
                    ctv = ctv + tev * rtv
                    qv = qv + rtv * rtv
                s_h, s_t = jnp.sum(sh), jnp.sum(st)
                a_h, a_t = jnp.sum(ah), jnp.sum(at_)
                c_h, c_t = jnp.sum(chv), jnp.sum(ctv)
                q = jnp.sum(qv)

                two = jnp.float32(2.0)
                eps = jnp.float32(1e-12)
                nh = a_h + two * s_h * c_h + s_h * s_h * q
                nt = a_t + two * s_t * c_t + s_t * s_t * q
                ih = _rsqrt(jnp.maximum(nh, eps))
                it = _rsqrt(jnp.maximum(nt, eps))

                acc = z
                for k, o in enumerate(CHUNK_OFFS):
                    sl = (w, pl.ds(o, L))
                    hev, tev = he[sl], te[sl]
                    rtv, rev = rt[sl], re[sl]
                    ph = (hev + s_h * rtv) * ih
                    pt = (tev + s_t * rtv) * it
                    term = jnp.abs(ph + rev - pt)
                    if k == len(CHUNK_OFFS) - 1:
                        term = term * tail_mask
                    acc = acc + term
                return jnp.where(lane_ids == w % L, jnp.sum(acc), dacc)

            for sub in range(W // L):
                dvec = lax.fori_loop(
                    0, L,
                    lambda w, a, _s=sub: _row(jnp.int32(_s * L) + w, a),
                    jnp.zeros((L,), jnp.float32))
                res[pl.ds(off + sub * L, L)] = dvec

        pltpu.sync_copy(res, out_hbm.at[pl.ds(base, rpt)])

    return dist_kernel


def kernel(ent_emb, rel_emb, ent_transfer, rel_transfer,
           pos_h_id, pos_t_id, pos_r_id, neg_h_id, neg_t_id, neg_r_id):
    b = pos_h_id.shape[0]
    h_id = jnp.concatenate([pos_h_id, neg_h_id]).astype(jnp.int32)
    t_id = jnp.concatenate([pos_t_id, neg_t_id]).astype(jnp.int32)
    r_id = jnp.concatenate([pos_r_id, neg_r_id]).astype(jnp.int32)
    dist = _build_dist_kernel(2 * b)(
        _pad_table(ent_emb), _pad_table(rel_emb),
        _pad_table(ent_transfer), _pad_table(rel_transfer),
        h_id, t_id, r_id)
    return dist[:b, None], dist[b:, None]
